# baseline (speedup 1.0000x reference)
import functools

import jax
import jax.numpy as jnp
from jax import lax
from jax.experimental import pallas as pl
from jax.experimental.pallas import tpu as pltpu
from jax.experimental.pallas import tpu_sc as plsc

PASS_TIME = 1.0
N_INTERVAL = 100
OUT_DIM = 64
NBINS = N_INTERVAL + 1
TSTRIDE = 65

NW = 32
DBLK = 8
B = 4096
LPAD = 208


def _tc_transpose(timestamp):
    batch, l1 = timestamp.shape
    blk = 1024

    def body(ts_ref, o_ref):
        t = jnp.transpose(ts_ref[...], (1, 0))
        o_ref[...] = jnp.concatenate(
            [t, jnp.zeros((LPAD - l1, blk), jnp.float32)], axis=0
        )

    return pl.pallas_call(
        body,
        grid=(batch // blk,),
        in_specs=[pl.BlockSpec((blk, l1), lambda i: (i, 0))],
        out_specs=pl.BlockSpec((LPAD, blk), lambda i: (0, i)),
        out_shape=jax.ShapeDtypeStruct((LPAD, batch), jnp.float32),
    )(timestamp)


def _sc_time_encode(tsT, table65, L):
    units = L * (OUT_DIM // DBLK)
    upw = units // NW
    tabn = NBINS * TSTRIDE + 3
    mesh = plsc.VectorSubcoreMesh(core_axis_name="c", subcore_axis_name="s")

    @functools.partial(
        pl.kernel,
        mesh=mesh,
        out_type=jax.ShapeDtypeStruct((L, OUT_DIM, B), jnp.float32),
        scratch_types=[
            pltpu.VMEM((B,), jnp.float32),
            pltpu.VMEM((B,), jnp.float32),
            pltpu.VMEM((tabn,), jnp.float32),
            pltpu.VMEM((DBLK, B), jnp.float32),
            pltpu.VMEM((DBLK, B), jnp.float32),
            pltpu.SemaphoreType.DMA,
            pltpu.SemaphoreType.DMA,
        ],
        compiler_params=pltpu.CompilerParams(needs_layout_passes=False),
    )
    def k(ts_h, tab_h, out_h, a_v, b_v, tab_v, bufa, bufb, sema, semb):
        wid = lax.axis_index("s") * 2 + lax.axis_index("c")
        u0 = wid * upw
        pltpu.sync_copy(tab_h, tab_v)
        iota = lax.iota(jnp.int32, 16)
        swz = [(iota + i) & (DBLK - 1) for i in range(DBLK)]

        def assemble(u, buf):
            l = u // DBLK
            d0 = pl.multiple_of((u % DBLK) * DBLK, DBLK)
            pltpu.sync_copy(ts_h.at[l], a_v)
            pltpu.sync_copy(ts_h.at[l + 1], b_v)
            tsl = tab_v.at[pl.ds(d0, tabn - 56)]

            def group(g, carry):
                s = pl.ds(pl.multiple_of(g * 16, 16), 16)
                dt = b_v[s] - a_v[s]
                q = (dt * (N_INTERVAL / PASS_TIME)).astype(jnp.int32)
                idx = jnp.minimum(jnp.maximum(q, 0), N_INTERVAL)
                idx65 = idx * TSTRIDE
                colv = iota + g * 16
                for i in range(DBLK):
                    val = plsc.load_gather(tsl, [idx65 + swz[i]])
                    plsc.store_scatter(buf, [swz[i], colv], val)
                return carry

            lax.fori_loop(0, B // 16, group, 0)
            return l, d0

        def start_store(l, d0, buf, sem):
            return pltpu.async_copy(
                buf, out_h.at[l, pl.ds(d0, DBLK), :], sem
            )

        def wait_store(buf, sem):
            pltpu.make_async_copy(
                buf, out_h.at[0, pl.ds(0, DBLK), :], sem
            ).wait()

        la, da = assemble(u0, bufa)
        start_store(la, da, bufa, sema)
        lb, db = assemble(u0 + 1, bufb)
        start_store(lb, db, bufb, semb)

        def pair(p, carry):
            u = u0 + 2 * p
            wait_store(bufa, sema)
            l1_, d1_ = assemble(u, bufa)
            start_store(l1_, d1_, bufa, sema)
            wait_store(bufb, semb)
            l2_, d2_ = assemble(u + 1, bufb)
            start_store(l2_, d2_, bufb, semb)
            return carry

        lax.fori_loop(1, upw // 2, pair, 0)
        wait_store(bufa, sema)
        wait_store(bufb, semb)

    return k(tsT, table65)


def kernel(inputs, timestamp, train, W, b):
    batch, L = inputs.shape
    table = W.T + b[None, :]
    t65 = jnp.concatenate(
        [table, jnp.zeros((NBINS, TSTRIDE - OUT_DIM), jnp.float32)], axis=1
    ).reshape(NBINS * TSTRIDE)
    t65 = jnp.concatenate([t65, jnp.zeros((3,), jnp.float32)])
    tsT = _tc_transpose(timestamp)
    out3 = _sc_time_encode(tsT, t65, L)
    emb = jnp.transpose(out3, (2, 0, 1))
    ts_prev = jnp.transpose(tsT[:L, :], (1, 0))
    return emb, ts_prev


# speedup vs baseline: 1.1261x; 1.1261x over previous
import functools

import jax
import jax.numpy as jnp
from jax import lax
from jax.experimental import pallas as pl
from jax.experimental.pallas import tpu as pltpu
from jax.experimental.pallas import tpu_sc as plsc

PASS_TIME = 1.0
N_INTERVAL = 100
OUT_DIM = 64
NBINS = N_INTERVAL + 1
TSTRIDE = 65

NW = 32
DBLK = 8
B = 4096
LPAD = 208


def _tc_transpose(timestamp):
    batch, l1 = timestamp.shape
    blk = 1024

    def body(ts_ref, o_ref):
        t = jnp.transpose(ts_ref[...], (1, 0))
        o_ref[...] = jnp.concatenate(
            [t, jnp.zeros((LPAD - l1, blk), jnp.float32)], axis=0
        )

    return pl.pallas_call(
        body,
        grid=(batch // blk,),
        in_specs=[pl.BlockSpec((blk, l1), lambda i: (i, 0))],
        out_specs=pl.BlockSpec((LPAD, blk), lambda i: (0, i)),
        out_shape=jax.ShapeDtypeStruct((LPAD, batch), jnp.float32),
    )(timestamp)


def _sc_time_encode(tsT, table65, L):
    units = L * (OUT_DIM // DBLK)
    upw = units // NW
    tabn = NBINS * TSTRIDE + 3
    mesh = plsc.VectorSubcoreMesh(core_axis_name="c", subcore_axis_name="s")

    @functools.partial(
        pl.kernel,
        mesh=mesh,
        out_type=jax.ShapeDtypeStruct((L, OUT_DIM, B), jnp.float32),
        scratch_types=[
            pltpu.VMEM((B,), jnp.float32),
            pltpu.VMEM((B,), jnp.float32),
            pltpu.VMEM((tabn,), jnp.float32),
            pltpu.VMEM((DBLK, B), jnp.float32),
            pltpu.VMEM((DBLK, B), jnp.float32),
            pltpu.SemaphoreType.DMA,
            pltpu.SemaphoreType.DMA,
        ],
        compiler_params=pltpu.CompilerParams(needs_layout_passes=False),
    )
    def k(ts_h, tab_h, out_h, a_v, b_v, tab_v, bufa, bufb, sema, semb):
        wid = lax.axis_index("s") * 2 + lax.axis_index("c")
        u0 = wid * upw
        pltpu.sync_copy(tab_h, tab_v)
        iota = lax.iota(jnp.int32, 16)
        swz = [(iota + i) & (DBLK - 1) for i in range(DBLK)]

        def assemble(u, buf):
            l = u // DBLK
            d0 = pl.multiple_of((u % DBLK) * DBLK, DBLK)

            @pl.when((u == u0) | (u % DBLK == 0))
            def _():
                pltpu.sync_copy(ts_h.at[l], a_v)
                pltpu.sync_copy(ts_h.at[l + 1], b_v)
            tsl = tab_v.at[pl.ds(d0, tabn - 56)]

            def group(g, carry):
                s = pl.ds(pl.multiple_of(g * 16, 16), 16)
                dt = b_v[s] - a_v[s]
                q = (dt * (N_INTERVAL / PASS_TIME)).astype(jnp.int32)
                idx = jnp.minimum(jnp.maximum(q, 0), N_INTERVAL)
                idx65 = idx * TSTRIDE
                colv = iota + g * 16
                for i in range(DBLK):
                    val = plsc.load_gather(tsl, [idx65 + swz[i]])
                    plsc.store_scatter(buf, [swz[i], colv], val)
                return carry

            lax.fori_loop(0, B // 16, group, 0)
            return l, d0

        def start_store(l, d0, buf, sem):
            return pltpu.async_copy(
                buf, out_h.at[l, pl.ds(d0, DBLK), :], sem
            )

        def wait_store(buf, sem):
            pltpu.make_async_copy(
                buf, out_h.at[0, pl.ds(0, DBLK), :], sem
            ).wait()

        la, da = assemble(u0, bufa)
        start_store(la, da, bufa, sema)
        lb, db = assemble(u0 + 1, bufb)
        start_store(lb, db, bufb, semb)

        def pair(p, carry):
            u = u0 + 2 * p
            wait_store(bufa, sema)
            l1_, d1_ = assemble(u, bufa)
            start_store(l1_, d1_, bufa, sema)
            wait_store(bufb, semb)
            l2_, d2_ = assemble(u + 1, bufb)
            start_store(l2_, d2_, bufb, semb)
            return carry

        lax.fori_loop(1, upw // 2, pair, 0)
        wait_store(bufa, sema)
        wait_store(bufb, semb)

    return k(tsT, table65)


def kernel(inputs, timestamp, train, W, b):
    batch, L = inputs.shape
    table = W.T + b[None, :]
    t65 = jnp.concatenate(
        [table, jnp.zeros((NBINS, TSTRIDE - OUT_DIM), jnp.float32)], axis=1
    ).reshape(NBINS * TSTRIDE)
    t65 = jnp.concatenate([t65, jnp.zeros((3,), jnp.float32)])
    tsT = _tc_transpose(timestamp)
    out3 = _sc_time_encode(tsT, t65, L)
    emb = jnp.transpose(out3, (2, 0, 1))
    ts_prev = jnp.transpose(tsT[:L, :], (1, 0))
    return emb, ts_prev


# speedup vs baseline: 1.3107x; 1.1639x over previous
import functools

import jax
import jax.numpy as jnp
from jax import lax
from jax.experimental import pallas as pl
from jax.experimental.pallas import tpu as pltpu
from jax.experimental.pallas import tpu_sc as plsc

PASS_TIME = 1.0
N_INTERVAL = 100
OUT_DIM = 64
NBINS = N_INTERVAL + 1
TSTRIDE = 65

NW = 32
DBLK = 8
B = 4096
LPAD = 208


def _tc_transpose(timestamp):
    batch, l1 = timestamp.shape
    blk = 1024

    def body(ts_ref, o_ref):
        t = jnp.transpose(ts_ref[...], (1, 0))
        o_ref[...] = jnp.concatenate(
            [t, jnp.zeros((LPAD - l1, blk), jnp.float32)], axis=0
        )

    return pl.pallas_call(
        body,
        grid=(batch // blk,),
        in_specs=[pl.BlockSpec((blk, l1), lambda i: (i, 0))],
        out_specs=pl.BlockSpec((LPAD, blk), lambda i: (0, i)),
        out_shape=jax.ShapeDtypeStruct((LPAD, batch), jnp.float32),
    )(timestamp)


def _sc_time_encode(tsT, table65, L):
    units = L * (OUT_DIM // DBLK)
    upw = units // NW
    tabn = NBINS * TSTRIDE + 3
    mesh = plsc.VectorSubcoreMesh(core_axis_name="c", subcore_axis_name="s")

    @functools.partial(
        pl.kernel,
        mesh=mesh,
        out_type=jax.ShapeDtypeStruct((L, OUT_DIM, B), jnp.float32),
        scratch_types=[
            pltpu.VMEM((B,), jnp.float32),
            pltpu.VMEM((B,), jnp.float32),
            pltpu.VMEM((B,), jnp.int32),
            pltpu.VMEM((tabn,), jnp.float32),
            pltpu.VMEM((DBLK, B), jnp.float32),
            pltpu.VMEM((DBLK, B), jnp.float32),
            pltpu.SemaphoreType.DMA,
            pltpu.SemaphoreType.DMA,
        ],
        compiler_params=pltpu.CompilerParams(needs_layout_passes=False),
    )
    def k(ts_h, tab_h, out_h, a_v, b_v, idx_v, tab_v, bufa, bufb, sema, semb):
        wid = lax.axis_index("s") * 2 + lax.axis_index("c")
        u0 = wid * upw
        pltpu.sync_copy(tab_h, tab_v)
        iota = lax.iota(jnp.int32, 16)
        swz = [(iota + i) & (DBLK - 1) for i in range(DBLK)]

        def assemble(u, buf):
            l = u // DBLK
            d0 = pl.multiple_of((u % DBLK) * DBLK, DBLK)

            @pl.when((u == u0) | (u % DBLK == 0))
            def _():
                pltpu.sync_copy(ts_h.at[l], a_v)
                pltpu.sync_copy(ts_h.at[l + 1], b_v)

                def mkidx(g, carry):
                    s = pl.ds(pl.multiple_of(g * 16, 16), 16)
                    dt = b_v[s] - a_v[s]
                    q = (dt * (N_INTERVAL / PASS_TIME)).astype(jnp.int32)
                    idx = jnp.minimum(jnp.maximum(q, 0), N_INTERVAL)
                    idx_v[s] = idx * TSTRIDE
                    return carry

                lax.fori_loop(0, B // 16, mkidx, 0)

            tsl = tab_v.at[pl.ds(d0, tabn - 56)]

            def group(g, carry):
                s = pl.ds(pl.multiple_of(g * 16, 16), 16)
                idx65 = idx_v[s]
                colv = iota + g * 16
                for i in range(DBLK):
                    val = plsc.load_gather(tsl, [idx65 + swz[i]])
                    plsc.store_scatter(buf, [swz[i], colv], val)
                return carry

            lax.fori_loop(0, B // 16, group, 0)
            return l, d0

        def start_store(l, d0, buf, sem):
            return pltpu.async_copy(
                buf, out_h.at[l, pl.ds(d0, DBLK), :], sem
            )

        def wait_store(buf, sem):
            pltpu.make_async_copy(
                buf, out_h.at[0, pl.ds(0, DBLK), :], sem
            ).wait()

        la, da = assemble(u0, bufa)
        start_store(la, da, bufa, sema)
        lb, db = assemble(u0 + 1, bufb)
        start_store(lb, db, bufb, semb)

        def pair(p, carry):
            u = u0 + 2 * p
            wait_store(bufa, sema)
            l1_, d1_ = assemble(u, bufa)
            start_store(l1_, d1_, bufa, sema)
            wait_store(bufb, semb)
            l2_, d2_ = assemble(u + 1, bufb)
            start_store(l2_, d2_, bufb, semb)
            return carry

        lax.fori_loop(1, upw // 2, pair, 0)
        wait_store(bufa, sema)
        wait_store(bufb, semb)

    return k(tsT, table65)


def kernel(inputs, timestamp, train, W, b):
    batch, L = inputs.shape
    table = W.T + b[None, :]
    t65 = jnp.concatenate(
        [table, jnp.zeros((NBINS, TSTRIDE - OUT_DIM), jnp.float32)], axis=1
    ).reshape(NBINS * TSTRIDE)
    t65 = jnp.concatenate([t65, jnp.zeros((3,), jnp.float32)])
    tsT = _tc_transpose(timestamp)
    out3 = _sc_time_encode(tsT, t65, L)
    emb = jnp.transpose(out3, (2, 0, 1))
    ts_prev = jnp.transpose(tsT[:L, :], (1, 0))
    return emb, ts_prev
